# baseline (speedup 1.0000x reference)
import functools

import jax
import jax.numpy as jnp
from jax import lax
from jax.experimental import pallas as pl
from jax.experimental.pallas import tpu as pltpu
from jax.experimental.pallas import tpu_sc as plsc

_N, _E, _D, _ED, _G = 10000, 320000, 128, 16, 64
_H1, _H2, _OUT = 256, 128, 1
_NC, _NS, _L = 2, 16, 16
_NW = _NC * _NS
_EPW = _E // _NW
_CH = 80
_NCHK = _EPW // _CH
_RPT = 624
_ZR = 208
_GCH = 40
_GNC = _EPW // _GCH
_SCH = 40
_SNC = _EPW // _SCH
_SIR = _SCH

_mesh = plsc.VectorSubcoreMesh(core_axis_name="c", subcore_axis_name="s")


@functools.partial(
    pl.kernel,
    out_type=jax.ShapeDtypeStruct((_E, 2 * _D), jnp.float32),
    mesh=_mesh,
    scratch_types=[
        pltpu.VMEM((_GNC, _GCH), jnp.int32),
        pltpu.VMEM((_GNC, _GCH), jnp.int32),
        pltpu.VMEM((_GCH, 2 * _D), jnp.float32),
        pltpu.VMEM((_GCH, 2 * _D), jnp.float32),
        pltpu.VMEM((_GCH, 2 * _D), jnp.float32),
        pltpu.VMEM((_GCH, 2 * _D), jnp.float32),
        pltpu.VMEM((_GCH, 2 * _D), jnp.float32),
        pltpu.VMEM((_GCH, 2 * _D), jnp.float32),
        pltpu.SemaphoreType.DMA,
        pltpu.SemaphoreType.DMA,
        pltpu.SemaphoreType.DMA,
        pltpu.SemaphoreType.DMA,
        pltpu.SemaphoreType.DMA,
        pltpu.SemaphoreType.DMA,
    ],
)
def _sc_gather(pd_hbm, ps_hbm, src3_hbm, dst3_hbm, a_hbm,
               di2, si2, gd0, gs0, ob0, gd1, gs1, ob1,
               sgd0, sgs0, swb0, sgd1, sgs1, swb1):
    c = lax.axis_index("c")
    s = lax.axis_index("s")
    wid = s * _NC + c
    base = wid * _EPW
    gd = (gd0, gd1)
    gs = (gs0, gs1)
    ob = (ob0, ob1)
    sgd = (sgd0, sgd1)
    sgs = (sgs0, sgs1)
    swb = (swb0, swb1)

    pltpu.sync_copy(dst3_hbm.at[wid], di2)
    pltpu.sync_copy(src3_hbm.at[wid], si2)

    def start_g(i, b):
        pltpu.async_copy(pd_hbm.at[di2.at[i]], gd[b], sgd[b])
        pltpu.async_copy(ps_hbm.at[si2.at[i]], gs[b], sgs[b])

    def wait_g(b):
        pltpu.make_async_copy(pd_hbm.at[di2.at[0]], gd[b], sgd[b]).wait()
        pltpu.make_async_copy(ps_hbm.at[si2.at[0]], gs[b], sgs[b]).wait()

    def wait_wb(b):
        pltpu.make_async_copy(ob[b], a_hbm.at[pl.ds(base, _GCH), :], swb[b]).wait()

    start_g(0, 0)
    start_g(1, 1)

    @pl.loop(0, _GNC // 2)
    def _pair(p):
        for b in (0, 1):
            i = 2 * p + b
            wait_g(b)

            @pl.when(p > 0)
            def _():
                wait_wb(b)

            def row(r, rc):
                for j in range(2 * _D // _L):
                    sl = pl.ds(j * _L, _L)
                    ob[b][r, sl] = gd[b][r, sl] + gs[b][r, sl]
                return rc

            lax.fori_loop(0, _GCH, row, 0)

            @pl.when(i + 2 < _GNC)
            def _():
                start_g(i + 2, b)

            pltpu.async_copy(ob[b], a_hbm.at[pl.ds(base + i * _GCH, _GCH), :],
                             swb[b])

    wait_wb(0)
    wait_wb(1)


@functools.partial(
    pl.kernel,
    out_type=jax.ShapeDtypeStruct((_NC, _N, _D), jnp.float32),
    mesh=_mesh,
    scratch_types=[
        pltpu.VMEM((_EPW // _SIR, _SIR), jnp.int32),
        pltpu.VMEM((_SCH, _D), jnp.float32),
        pltpu.VMEM((_SCH, _D), jnp.float32),
        pltpu.VMEM_SHARED((_N, _D), jnp.float32),
        pltpu.SemaphoreType.DMA,
        pltpu.SemaphoreType.DMA,
        pltpu.SemaphoreType.DMA,
        pltpu.SemaphoreType.DMA,
    ],
)
def _sc_scatter(m_hbm, dsts_hbm, zeros_hbm, out_hbm, di2, mb0, mb1, acc,
                sm0, sm1, ssc0, ssc1):
    c = lax.axis_index("c")
    s = lax.axis_index("s")
    wid = s * _NC + c
    base = wid * _EPW
    mb = (mb0, mb1)
    sm = (sm0, sm1)
    ssc = (ssc0, ssc1)

    pltpu.sync_copy(dsts_hbm.at[wid], di2)

    r_base = s * _RPT
    pltpu.sync_copy(zeros_hbm.at[pl.ds(r_base, _RPT), :],
                    acc.at[pl.ds(r_base, _RPT), :])

    @pl.when(s == _NS - 1)
    def _():
        pltpu.sync_copy(zeros_hbm.at[pl.ds(_NS * _RPT, _N - _NS * _RPT), :],
                        acc.at[pl.ds(_NS * _RPT, _N - _NS * _RPT), :])

    plsc.subcore_barrier()

    def start_m(i, b):
        pltpu.async_copy(m_hbm.at[pl.ds(base + i * _SCH, _SCH), :], mb[b], sm[b])

    def wait_m(b):
        pltpu.make_async_copy(m_hbm.at[pl.ds(base, _SCH), :], mb[b], sm[b]).wait()

    def start_sc(i, b):
        pltpu.async_copy(mb[b], acc.at[di2.at[i]], ssc[b], add=True)

    def wait_sc(b):
        pltpu.make_async_copy(mb[b], acc.at[di2.at[0]], ssc[b]).wait()

    start_m(0, 0)
    start_m(1, 1)

    @pl.loop(0, _SNC // 2)
    def _pair(p):
        for b in (0, 1):
            wait_m(b)
            start_sc(2 * p + b, b)
        for b in (0, 1):
            i = 2 * p + b
            wait_sc(b)

            @pl.when(i + 2 < _SNC)
            def _():
                start_m(i + 2, b)

    plsc.subcore_barrier()
    pltpu.sync_copy(acc.at[pl.ds(r_base, _RPT), :],
                    out_hbm.at[c, pl.ds(r_base, _RPT), :])

    @pl.when(s == _NS - 1)
    def _():
        pltpu.sync_copy(acc.at[pl.ds(_NS * _RPT, _N - _NS * _RPT), :],
                        out_hbm.at[c, pl.ds(_NS * _RPT, _N - _NS * _RPT), :])


_BE = 2000


def _gate_body(a_ref, ea_ref, wef_ref, wes_ref, bf_ref, bs_ref, m_ref):
    a = a_ref[...]
    ea = ea_ref[...]
    zf = a[:, :_D] + jnp.dot(ea, wef_ref[...],
                             preferred_element_type=jnp.float32, precision=lax.Precision.HIGHEST) + bf_ref[...]
    zs = a[:, _D:] + jnp.dot(ea, wes_ref[...],
                             preferred_element_type=jnp.float32, precision=lax.Precision.HIGHEST) + bs_ref[...]
    m_ref[...] = jax.nn.sigmoid(zf) * jax.nn.softplus(zs)


def _gate(a, ea, wef, wes, bf, bs):
    grid = (_E // _BE,)
    return pl.pallas_call(
        _gate_body,
        grid=grid,
        in_specs=[
            pl.BlockSpec((_BE, 2 * _D), lambda i: (i, 0)),
            pl.BlockSpec((_BE, _ED), lambda i: (i, 0)),
            pl.BlockSpec((_ED, _D), lambda i: (0, 0)),
            pl.BlockSpec((_ED, _D), lambda i: (0, 0)),
            pl.BlockSpec((1, _D), lambda i: (0, 0)),
            pl.BlockSpec((1, _D), lambda i: (0, 0)),
        ],
        out_specs=pl.BlockSpec((_BE, _D), lambda i: (i, 0)),
        out_shape=jax.ShapeDtypeStruct((_E, _D), jnp.float32),
    )(a, ea, wef, wes, bf, bs)


_BN_ROWS = 2000


def _proj_body(h_ref, wd_ref, ws_ref, pd_ref, ps_ref):
    h = h_ref[...]
    pd_ref[...] = jnp.dot(h, wd_ref[...], preferred_element_type=jnp.float32)
    ps_ref[...] = jnp.dot(h, ws_ref[...], preferred_element_type=jnp.float32)


def _projs(h, wd, ws):
    grid = (_N // _BN_ROWS,)
    return pl.pallas_call(
        _proj_body,
        grid=grid,
        in_specs=[
            pl.BlockSpec((_BN_ROWS, _D), lambda i: (i, 0)),
            pl.BlockSpec((_D, 2 * _D), lambda i: (0, 0)),
            pl.BlockSpec((_D, 2 * _D), lambda i: (0, 0)),
        ],
        out_specs=[
            pl.BlockSpec((_BN_ROWS, 2 * _D), lambda i: (i, 0)),
            pl.BlockSpec((_BN_ROWS, 2 * _D), lambda i: (i, 0)),
        ],
        out_shape=[
            jax.ShapeDtypeStruct((_N, 2 * _D), jnp.float32),
            jax.ShapeDtypeStruct((_N, 2 * _D), jnp.float32),
        ],
    )(h, wd, ws)


def _bn(h, g, b):
    mu = jnp.mean(h, axis=0, keepdims=True)
    var = jnp.mean((h - mu) * (h - mu), axis=0, keepdims=True)
    return (h - mu) / jnp.sqrt(var + 1e-5) * g + b


def _bnres_body(p_ref, h_ref, g_ref, b_ref, hn_ref):
    agg = p_ref[0] + p_ref[1]
    hn_ref[...] = _bn(agg, g_ref[...], b_ref[...]) + h_ref[...]


def _bnres(parts, h, g, b):
    return pl.pallas_call(
        _bnres_body,
        in_specs=[
            pl.BlockSpec((2, _N, _D), lambda: (0, 0, 0)),
            pl.BlockSpec((_N, _D), lambda: (0, 0)),
            pl.BlockSpec((1, _D), lambda: (0, 0)),
            pl.BlockSpec((1, _D), lambda: (0, 0)),
        ],
        out_specs=pl.BlockSpec((_N, _D), lambda: (0, 0)),
        out_shape=jax.ShapeDtypeStruct((_N, _D), jnp.float32),
    )(parts, h, g, b)


def _head_body(p_ref, h_ref, g_ref, b_ref, batch_ref,
               w1_ref, b1_ref, g1_ref, be1_ref,
               w2_ref, b2_ref, g2_ref, be2_ref,
               w3_ref, b3_ref, out_ref):
    agg = p_ref[0] + p_ref[1]
    h4 = _bn(agg, g_ref[...], b_ref[...]) + h_ref[...]
    bvec = batch_ref[...]
    gids = lax.broadcasted_iota(jnp.int32, (_G, 1), 0)
    onehot = (bvec == gids).astype(jnp.float32)
    sums = jnp.dot(onehot, h4, preferred_element_type=jnp.float32, precision=lax.Precision.HIGHEST)
    cnt = jnp.sum(onehot, axis=1, keepdims=True)
    pooled = sums / jnp.maximum(cnt, 1.0)
    z1 = jnp.dot(pooled, w1_ref[...], preferred_element_type=jnp.float32, precision=lax.Precision.HIGHEST) + b1_ref[...]
    a1 = jnp.maximum(_bn(z1, g1_ref[...], be1_ref[...]), 0.0)
    z2 = jnp.dot(a1, w2_ref[...], preferred_element_type=jnp.float32, precision=lax.Precision.HIGHEST) + b2_ref[...]
    a2 = jnp.maximum(_bn(z2, g2_ref[...], be2_ref[...]), 0.0)
    out_ref[...] = jnp.dot(a2, w3_ref[...], preferred_element_type=jnp.float32, precision=lax.Precision.HIGHEST) + b3_ref[...]


def _head(parts, h, g, b, batch2d, w1, b1, g1, be1, w2, b2, g2, be2, w3, b3):
    full = lambda s: pl.BlockSpec(s, lambda: tuple(0 for _ in s))
    return pl.pallas_call(
        _head_body,
        in_specs=[
            pl.BlockSpec((2, _N, _D), lambda: (0, 0, 0)),
            full((_N, _D)), full((1, _D)), full((1, _D)), full((1, _N)),
            full((_D, _H1)), full((1, _H1)), full((1, _H1)), full((1, _H1)),
            full((_H1, _H2)), full((1, _H2)), full((1, _H2)), full((1, _H2)),
            full((_H2, _OUT)), full((1, _OUT)),
        ],
        out_specs=full((_G, _OUT)),
        out_shape=jax.ShapeDtypeStruct((_G, _OUT), jnp.float32),
    )(parts, h, g, b,
      batch2d, w1, b1, g1, be1, w2, b2, g2, be2, w3, b3)


def kernel(x, edge_index, edge_attr, batch, Wf, bf, Ws, bs, bng, bnb,
           W1, b1, g1, be1, W2, b2, g2, be2, W3, b3):
    src3 = edge_index[0].reshape(_NW, _GNC, _GCH)
    dst3 = edge_index[1].reshape(_NW, _GNC, _GCH)
    dsts = edge_index[1].reshape(_NW, _EPW // _SIR, _SIR)
    zeros = jnp.zeros((_N, _D), jnp.float32)
    batch2d = batch.reshape(1, _N)

    wd = [jnp.concatenate([Wf[i, :_D], Ws[i, :_D]], axis=1) for i in range(4)]
    wsrc = [jnp.concatenate([Wf[i, _D:2 * _D], Ws[i, _D:2 * _D]], axis=1)
            for i in range(4)]
    wef = [Wf[i, 2 * _D:] for i in range(4)]
    wes = [Ws[i, 2 * _D:] for i in range(4)]
    bf2 = [bf[i].reshape(1, _D) for i in range(4)]
    bs2 = [bs[i].reshape(1, _D) for i in range(4)]
    g2d = [bng[i].reshape(1, _D) for i in range(4)]
    b2d = [bnb[i].reshape(1, _D) for i in range(4)]

    h = x
    pd, ps = _projs(x, wd[0], wsrc[0])
    out = None
    for i in range(4):
        a = _sc_gather(pd, ps, src3, dst3)
        m = _gate(a, edge_attr, wef[i], wes[i], bf2[i], bs2[i])
        parts = _sc_scatter(m, dsts, zeros)
        if i < 3:
            h = _bnres(parts, h, g2d[i], b2d[i])
            pd, ps = _projs(h, wd[i + 1], wsrc[i + 1])
        else:
            out = _head(parts, h, g2d[i], b2d[i], batch2d,
                        W1, b1.reshape(1, _H1), g1.reshape(1, _H1),
                        be1.reshape(1, _H1),
                        W2, b2.reshape(1, _H2), g2.reshape(1, _H2),
                        be2.reshape(1, _H2),
                        W3, b3.reshape(1, _OUT))
    return out


# speedup vs baseline: 1.0007x; 1.0007x over previous
import functools

import jax
import jax.numpy as jnp
from jax import lax
from jax.experimental import pallas as pl
from jax.experimental.pallas import tpu as pltpu
from jax.experimental.pallas import tpu_sc as plsc

_N, _E, _D, _ED, _G = 10000, 320000, 128, 16, 64
_H1, _H2, _OUT = 256, 128, 1
_NC, _NS, _L = 2, 16, 16
_NW = _NC * _NS
_EPW = _E // _NW
_CH = 80
_NCHK = _EPW // _CH
_RPT = 624
_ZR = 208
_GCH = 40
_GNC = _EPW // _GCH
_SCH = 40
_SNC = _EPW // _SCH
_SIR = _SCH

_mesh = plsc.VectorSubcoreMesh(core_axis_name="c", subcore_axis_name="s")


@functools.partial(
    pl.kernel,
    out_type=jax.ShapeDtypeStruct((_E, 2 * _D), jnp.float32),
    mesh=_mesh,
    scratch_types=[
        pltpu.VMEM((_GNC, _GCH), jnp.int32),
        pltpu.VMEM((_GNC, _GCH), jnp.int32),
        pltpu.VMEM((_GCH, 2 * _D), jnp.float32),
        pltpu.VMEM((_GCH, 2 * _D), jnp.float32),
        pltpu.VMEM((_GCH, 2 * _D), jnp.float32),
        pltpu.VMEM((_GCH, 2 * _D), jnp.float32),
        pltpu.VMEM((_GCH, 2 * _D), jnp.float32),
        pltpu.VMEM((_GCH, 2 * _D), jnp.float32),
        pltpu.SemaphoreType.DMA,
        pltpu.SemaphoreType.DMA,
        pltpu.SemaphoreType.DMA,
        pltpu.SemaphoreType.DMA,
        pltpu.SemaphoreType.DMA,
        pltpu.SemaphoreType.DMA,
    ],
)
def _sc_gather(pd_hbm, ps_hbm, src3_hbm, dst3_hbm, a_hbm,
               di2, si2, gd0, gs0, ob0, gd1, gs1, ob1,
               sgd0, sgs0, swb0, sgd1, sgs1, swb1):
    c = lax.axis_index("c")
    s = lax.axis_index("s")
    wid = s * _NC + c
    base = wid * _EPW
    gd = (gd0, gd1)
    gs = (gs0, gs1)
    ob = (ob0, ob1)
    sgd = (sgd0, sgd1)
    sgs = (sgs0, sgs1)
    swb = (swb0, swb1)

    pltpu.sync_copy(dst3_hbm.at[wid], di2)
    pltpu.sync_copy(src3_hbm.at[wid], si2)

    def start_g(i, b):
        pltpu.async_copy(pd_hbm.at[di2.at[i]], gd[b], sgd[b])
        pltpu.async_copy(ps_hbm.at[si2.at[i]], gs[b], sgs[b])

    def wait_g(b):
        pltpu.make_async_copy(pd_hbm.at[di2.at[0]], gd[b], sgd[b]).wait()
        pltpu.make_async_copy(ps_hbm.at[si2.at[0]], gs[b], sgs[b]).wait()

    def wait_wb(b):
        pltpu.make_async_copy(ob[b], a_hbm.at[pl.ds(base, _GCH), :], swb[b]).wait()

    start_g(0, 0)
    start_g(1, 1)

    @pl.loop(0, _GNC // 2)
    def _pair(p):
        for b in (0, 1):
            i = 2 * p + b
            wait_g(b)

            @pl.when(p > 0)
            def _():
                wait_wb(b)

            def row(r, rc):
                for j in range(2 * _D // _L):
                    sl = pl.ds(j * _L, _L)
                    ob[b][r, sl] = gd[b][r, sl] + gs[b][r, sl]
                return rc

            lax.fori_loop(0, _GCH, row, 0)

            @pl.when(i + 2 < _GNC)
            def _():
                start_g(i + 2, b)

            pltpu.async_copy(ob[b], a_hbm.at[pl.ds(base + i * _GCH, _GCH), :],
                             swb[b])

    wait_wb(0)
    wait_wb(1)


@functools.partial(
    pl.kernel,
    out_type=jax.ShapeDtypeStruct((_NC, _N, _D), jnp.float32),
    mesh=_mesh,
    scratch_types=[
        pltpu.VMEM((_EPW // _SIR, _SIR), jnp.int32),
        pltpu.VMEM((_SCH, _D), jnp.float32),
        pltpu.VMEM((_SCH, _D), jnp.float32),
        pltpu.VMEM_SHARED((_N, _D), jnp.float32),
        pltpu.SemaphoreType.DMA,
        pltpu.SemaphoreType.DMA,
        pltpu.SemaphoreType.DMA,
        pltpu.SemaphoreType.DMA,
    ],
)
def _sc_scatter(m_hbm, dsts_hbm, zeros_hbm, out_hbm, di2, mb0, mb1, acc,
                sm0, sm1, ssc0, ssc1):
    c = lax.axis_index("c")
    s = lax.axis_index("s")
    wid = s * _NC + c
    base = wid * _EPW
    mb = (mb0, mb1)
    sm = (sm0, sm1)
    ssc = (ssc0, ssc1)

    pltpu.sync_copy(dsts_hbm.at[wid], di2)

    r_base = s * _RPT
    pltpu.sync_copy(zeros_hbm.at[pl.ds(r_base, _RPT), :],
                    acc.at[pl.ds(r_base, _RPT), :])

    @pl.when(s == _NS - 1)
    def _():
        pltpu.sync_copy(zeros_hbm.at[pl.ds(_NS * _RPT, _N - _NS * _RPT), :],
                        acc.at[pl.ds(_NS * _RPT, _N - _NS * _RPT), :])

    plsc.subcore_barrier()

    def start_m(i, b):
        pltpu.async_copy(m_hbm.at[pl.ds(base + i * _SCH, _SCH), :], mb[b], sm[b])

    def wait_m(b):
        pltpu.make_async_copy(m_hbm.at[pl.ds(base, _SCH), :], mb[b], sm[b]).wait()

    def start_sc(i, b):
        pltpu.async_copy(mb[b], acc.at[di2.at[i]], ssc[b], add=True)

    def wait_sc(b):
        pltpu.make_async_copy(mb[b], acc.at[di2.at[0]], ssc[b]).wait()

    start_m(0, 0)
    start_m(1, 1)

    @pl.loop(0, _SNC // 2)
    def _pair(p):
        for b in (0, 1):
            wait_m(b)
            start_sc(2 * p + b, b)
        for b in (0, 1):
            i = 2 * p + b
            wait_sc(b)

            @pl.when(i + 2 < _SNC)
            def _():
                start_m(i + 2, b)

    plsc.subcore_barrier()
    pltpu.sync_copy(acc.at[pl.ds(r_base, _RPT), :],
                    out_hbm.at[c, pl.ds(r_base, _RPT), :])

    @pl.when(s == _NS - 1)
    def _():
        pltpu.sync_copy(acc.at[pl.ds(_NS * _RPT, _N - _NS * _RPT), :],
                        out_hbm.at[c, pl.ds(_NS * _RPT, _N - _NS * _RPT), :])


_BE = 2000


def _gate_body(a_ref, ea_ref, wef_ref, wes_ref, bf_ref, bs_ref, m_ref):
    a = a_ref[...]
    ea = ea_ref[...]
    zf = a[:, :_D] + jnp.dot(ea, wef_ref[...],
                             preferred_element_type=jnp.float32, precision=lax.Precision.HIGHEST) + bf_ref[...]
    zs = a[:, _D:] + jnp.dot(ea, wes_ref[...],
                             preferred_element_type=jnp.float32, precision=lax.Precision.HIGHEST) + bs_ref[...]
    m_ref[...] = jax.nn.sigmoid(zf) * jax.nn.softplus(zs)


def _gate(a, ea, wef, wes, bf, bs):
    grid = (_E // _BE,)
    return pl.pallas_call(
        _gate_body,
        grid=grid,
        in_specs=[
            pl.BlockSpec((_BE, 2 * _D), lambda i: (i, 0)),
            pl.BlockSpec((_BE, _ED), lambda i: (i, 0)),
            pl.BlockSpec((_ED, _D), lambda i: (0, 0)),
            pl.BlockSpec((_ED, _D), lambda i: (0, 0)),
            pl.BlockSpec((1, _D), lambda i: (0, 0)),
            pl.BlockSpec((1, _D), lambda i: (0, 0)),
        ],
        out_specs=pl.BlockSpec((_BE, _D), lambda i: (i, 0)),
        out_shape=jax.ShapeDtypeStruct((_E, _D), jnp.float32),
    )(a, ea, wef, wes, bf, bs)


_BN_ROWS = 2000


def _proj_body(h_ref, wd_ref, ws_ref, pd_ref, ps_ref):
    h = h_ref[...]
    pd_ref[...] = jnp.dot(h, wd_ref[...], preferred_element_type=jnp.float32)
    ps_ref[...] = jnp.dot(h, ws_ref[...], preferred_element_type=jnp.float32)


def _projs(h, wd, ws):
    grid = (_N // _BN_ROWS,)
    return pl.pallas_call(
        _proj_body,
        grid=grid,
        in_specs=[
            pl.BlockSpec((_BN_ROWS, _D), lambda i: (i, 0)),
            pl.BlockSpec((_D, 2 * _D), lambda i: (0, 0)),
            pl.BlockSpec((_D, 2 * _D), lambda i: (0, 0)),
        ],
        out_specs=[
            pl.BlockSpec((_BN_ROWS, 2 * _D), lambda i: (i, 0)),
            pl.BlockSpec((_BN_ROWS, 2 * _D), lambda i: (i, 0)),
        ],
        out_shape=[
            jax.ShapeDtypeStruct((_N, 2 * _D), jnp.float32),
            jax.ShapeDtypeStruct((_N, 2 * _D), jnp.float32),
        ],
    )(h, wd, ws)


def _bn(h, g, b):
    mu = jnp.mean(h, axis=0, keepdims=True)
    var = jnp.mean((h - mu) * (h - mu), axis=0, keepdims=True)
    return (h - mu) / jnp.sqrt(var + 1e-5) * g + b


def _bnproj_body(p_ref, h_ref, g_ref, b_ref, wd_ref, ws_ref,
                 hn_ref, pd_ref, ps_ref):
    agg = p_ref[0] + p_ref[1]
    hn = _bn(agg, g_ref[...], b_ref[...]) + h_ref[...]
    hn_ref[...] = hn
    pd_ref[...] = jnp.dot(hn, wd_ref[...], preferred_element_type=jnp.float32)
    ps_ref[...] = jnp.dot(hn, ws_ref[...], preferred_element_type=jnp.float32)


def _bnproj(parts, h, g, b, wd, ws):
    return pl.pallas_call(
        _bnproj_body,
        in_specs=[
            pl.BlockSpec((2, _N, _D), lambda: (0, 0, 0)),
            pl.BlockSpec((_N, _D), lambda: (0, 0)),
            pl.BlockSpec((1, _D), lambda: (0, 0)),
            pl.BlockSpec((1, _D), lambda: (0, 0)),
            pl.BlockSpec((_D, 2 * _D), lambda: (0, 0)),
            pl.BlockSpec((_D, 2 * _D), lambda: (0, 0)),
        ],
        out_specs=[
            pl.BlockSpec((_N, _D), lambda: (0, 0)),
            pl.BlockSpec((_N, 2 * _D), lambda: (0, 0)),
            pl.BlockSpec((_N, 2 * _D), lambda: (0, 0)),
        ],
        out_shape=[
            jax.ShapeDtypeStruct((_N, _D), jnp.float32),
            jax.ShapeDtypeStruct((_N, 2 * _D), jnp.float32),
            jax.ShapeDtypeStruct((_N, 2 * _D), jnp.float32),
        ],
    )(parts, h, g, b, wd, ws)


def _head_body(p_ref, h_ref, g_ref, b_ref, batch_ref,
               w1_ref, b1_ref, g1_ref, be1_ref,
               w2_ref, b2_ref, g2_ref, be2_ref,
               w3_ref, b3_ref, out_ref):
    agg = p_ref[0] + p_ref[1]
    h4 = _bn(agg, g_ref[...], b_ref[...]) + h_ref[...]
    bvec = batch_ref[...]
    gids = lax.broadcasted_iota(jnp.int32, (_G, 1), 0)
    onehot = (bvec == gids).astype(jnp.float32)
    sums = jnp.dot(onehot, h4, preferred_element_type=jnp.float32, precision=lax.Precision.HIGHEST)
    cnt = jnp.sum(onehot, axis=1, keepdims=True)
    pooled = sums / jnp.maximum(cnt, 1.0)
    z1 = jnp.dot(pooled, w1_ref[...], preferred_element_type=jnp.float32, precision=lax.Precision.HIGHEST) + b1_ref[...]
    a1 = jnp.maximum(_bn(z1, g1_ref[...], be1_ref[...]), 0.0)
    z2 = jnp.dot(a1, w2_ref[...], preferred_element_type=jnp.float32, precision=lax.Precision.HIGHEST) + b2_ref[...]
    a2 = jnp.maximum(_bn(z2, g2_ref[...], be2_ref[...]), 0.0)
    out_ref[...] = jnp.dot(a2, w3_ref[...], preferred_element_type=jnp.float32, precision=lax.Precision.HIGHEST) + b3_ref[...]


def _head(parts, h, g, b, batch2d, w1, b1, g1, be1, w2, b2, g2, be2, w3, b3):
    full = lambda s: pl.BlockSpec(s, lambda: tuple(0 for _ in s))
    return pl.pallas_call(
        _head_body,
        in_specs=[
            pl.BlockSpec((2, _N, _D), lambda: (0, 0, 0)),
            full((_N, _D)), full((1, _D)), full((1, _D)), full((1, _N)),
            full((_D, _H1)), full((1, _H1)), full((1, _H1)), full((1, _H1)),
            full((_H1, _H2)), full((1, _H2)), full((1, _H2)), full((1, _H2)),
            full((_H2, _OUT)), full((1, _OUT)),
        ],
        out_specs=full((_G, _OUT)),
        out_shape=jax.ShapeDtypeStruct((_G, _OUT), jnp.float32),
    )(parts, h, g, b,
      batch2d, w1, b1, g1, be1, w2, b2, g2, be2, w3, b3)


def kernel(x, edge_index, edge_attr, batch, Wf, bf, Ws, bs, bng, bnb,
           W1, b1, g1, be1, W2, b2, g2, be2, W3, b3):
    src3 = edge_index[0].reshape(_NW, _GNC, _GCH)
    dst3 = edge_index[1].reshape(_NW, _GNC, _GCH)
    dsts = edge_index[1].reshape(_NW, _EPW // _SIR, _SIR)
    zeros = jnp.zeros((_N, _D), jnp.float32)
    batch2d = batch.reshape(1, _N)

    wd = [jnp.concatenate([Wf[i, :_D], Ws[i, :_D]], axis=1) for i in range(4)]
    wsrc = [jnp.concatenate([Wf[i, _D:2 * _D], Ws[i, _D:2 * _D]], axis=1)
            for i in range(4)]
    wef = [Wf[i, 2 * _D:] for i in range(4)]
    wes = [Ws[i, 2 * _D:] for i in range(4)]
    bf2 = [bf[i].reshape(1, _D) for i in range(4)]
    bs2 = [bs[i].reshape(1, _D) for i in range(4)]
    g2d = [bng[i].reshape(1, _D) for i in range(4)]
    b2d = [bnb[i].reshape(1, _D) for i in range(4)]

    h = x
    pd, ps = _projs(x, wd[0], wsrc[0])
    out = None
    for i in range(4):
        a = _sc_gather(pd, ps, src3, dst3)
        m = _gate(a, edge_attr, wef[i], wes[i], bf2[i], bs2[i])
        parts = _sc_scatter(m, dsts, zeros)
        if i < 3:
            h, pd, ps = _bnproj(parts, h, g2d[i], b2d[i], wd[i + 1], wsrc[i + 1])
        else:
            out = _head(parts, h, g2d[i], b2d[i], batch2d,
                        W1, b1.reshape(1, _H1), g1.reshape(1, _H1),
                        be1.reshape(1, _H1),
                        W2, b2.reshape(1, _H2), g2.reshape(1, _H2),
                        be2.reshape(1, _H2),
                        W3, b3.reshape(1, _OUT))
    return out


# speedup vs baseline: 1.1252x; 1.1244x over previous
import functools

import jax
import jax.numpy as jnp
from jax import lax
from jax.experimental import pallas as pl
from jax.experimental.pallas import tpu as pltpu
from jax.experimental.pallas import tpu_sc as plsc

_N, _E, _D, _ED, _G = 10000, 320000, 128, 16, 64
_H1, _H2, _OUT = 256, 128, 1
_NC, _NS, _L = 2, 16, 16
_NW = _NC * _NS
_EPW = _E // _NW
_CH = 80
_NCHK = _EPW // _CH
_RPT = 624
_ZR = 208
_GCH = 40
_GNC = _EPW // _GCH
_SCH = 40
_SNC = _EPW // _SCH
_SIR = _SCH

_mesh = plsc.VectorSubcoreMesh(core_axis_name="c", subcore_axis_name="s")


@functools.partial(
    pl.kernel,
    out_type=jax.ShapeDtypeStruct((_E, 2 * _D), jnp.float32),
    mesh=_mesh,
    scratch_types=[
        pltpu.VMEM((_GNC, _GCH), jnp.int32),
        pltpu.VMEM((_GNC, _GCH), jnp.int32),
        pltpu.VMEM((_GCH, 2 * _D), jnp.float32),
        pltpu.VMEM((_GCH, 2 * _D), jnp.float32),
        pltpu.VMEM((_GCH, 2 * _D), jnp.float32),
        pltpu.VMEM((_GCH, 2 * _D), jnp.float32),
        pltpu.VMEM((_GCH, 2 * _D), jnp.float32),
        pltpu.VMEM((_GCH, 2 * _D), jnp.float32),
        pltpu.SemaphoreType.DMA,
        pltpu.SemaphoreType.DMA,
        pltpu.SemaphoreType.DMA,
        pltpu.SemaphoreType.DMA,
        pltpu.SemaphoreType.DMA,
        pltpu.SemaphoreType.DMA,
    ],
)
def _sc_gather(pd_hbm, ps_hbm, src3_hbm, dst3_hbm, a_hbm,
               di2, si2, gd0, gs0, ob0, gd1, gs1, ob1,
               sgd0, sgs0, swb0, sgd1, sgs1, swb1):
    c = lax.axis_index("c")
    s = lax.axis_index("s")
    wid = s * _NC + c
    base = wid * _EPW
    gd = (gd0, gd1)
    gs = (gs0, gs1)
    ob = (ob0, ob1)
    sgd = (sgd0, sgd1)
    sgs = (sgs0, sgs1)
    swb = (swb0, swb1)

    pltpu.sync_copy(dst3_hbm.at[wid], di2)
    pltpu.sync_copy(src3_hbm.at[wid], si2)

    def start_g(i, b):
        pltpu.async_copy(pd_hbm.at[di2.at[i]], gd[b], sgd[b])
        pltpu.async_copy(ps_hbm.at[si2.at[i]], gs[b], sgs[b])

    def wait_g(b):
        pltpu.make_async_copy(pd_hbm.at[di2.at[0]], gd[b], sgd[b]).wait()
        pltpu.make_async_copy(ps_hbm.at[si2.at[0]], gs[b], sgs[b]).wait()

    def wait_wb(b):
        pltpu.make_async_copy(ob[b], a_hbm.at[pl.ds(base, _GCH), :], swb[b]).wait()

    start_g(0, 0)
    start_g(1, 1)

    @pl.loop(0, _GNC // 2)
    def _pair(p):
        for b in (0, 1):
            i = 2 * p + b
            wait_g(b)

            @pl.when(p > 0)
            def _():
                wait_wb(b)

            def row(r, rc):
                for j in range(2 * _D // _L):
                    sl = pl.ds(j * _L, _L)
                    ob[b][r, sl] = gd[b][r, sl] + gs[b][r, sl]
                return rc

            lax.fori_loop(0, _GCH, row, 0)

            @pl.when(i + 2 < _GNC)
            def _():
                start_g(i + 2, b)

            pltpu.async_copy(ob[b], a_hbm.at[pl.ds(base + i * _GCH, _GCH), :],
                             swb[b])

    wait_wb(0)
    wait_wb(1)


@functools.partial(
    pl.kernel,
    out_type=jax.ShapeDtypeStruct((_NC, _N, _D), jnp.float32),
    mesh=_mesh,
    scratch_types=[
        pltpu.VMEM((_EPW // _SIR, _SIR), jnp.int32),
        pltpu.VMEM((_SCH, _D), jnp.float32),
        pltpu.VMEM((_SCH, _D), jnp.float32),
        pltpu.VMEM_SHARED((_N, _D), jnp.float32),
        pltpu.SemaphoreType.DMA,
        pltpu.SemaphoreType.DMA,
        pltpu.SemaphoreType.DMA,
        pltpu.SemaphoreType.DMA,
    ],
)
def _sc_scatter(m_hbm, dsts_hbm, zeros_hbm, out_hbm, di2, mb0, mb1, acc,
                sm0, sm1, ssc0, ssc1):
    c = lax.axis_index("c")
    s = lax.axis_index("s")
    wid = s * _NC + c
    base = wid * _EPW
    mb = (mb0, mb1)
    sm = (sm0, sm1)
    ssc = (ssc0, ssc1)

    pltpu.sync_copy(dsts_hbm.at[wid], di2)

    r_base = s * _RPT
    pltpu.sync_copy(zeros_hbm.at[pl.ds(r_base, _RPT), :],
                    acc.at[pl.ds(r_base, _RPT), :])

    @pl.when(s == _NS - 1)
    def _():
        pltpu.sync_copy(zeros_hbm.at[pl.ds(_NS * _RPT, _N - _NS * _RPT), :],
                        acc.at[pl.ds(_NS * _RPT, _N - _NS * _RPT), :])

    plsc.subcore_barrier()

    def start_m(i, b):
        pltpu.async_copy(m_hbm.at[pl.ds(base + i * _SCH, _SCH), :], mb[b], sm[b])

    def wait_m(b):
        pltpu.make_async_copy(m_hbm.at[pl.ds(base, _SCH), :], mb[b], sm[b]).wait()

    def start_sc(i, b):
        pltpu.async_copy(mb[b], acc.at[di2.at[i]], ssc[b], add=True)

    def wait_sc(b):
        pltpu.make_async_copy(mb[b], acc.at[di2.at[0]], ssc[b]).wait()

    start_m(0, 0)
    start_m(1, 1)

    @pl.loop(0, _SNC // 2)
    def _pair(p):
        for b in (0, 1):
            wait_m(b)
            start_sc(2 * p + b, b)
        for b in (0, 1):
            i = 2 * p + b
            wait_sc(b)

            @pl.when(i + 2 < _SNC)
            def _():
                start_m(i + 2, b)

    plsc.subcore_barrier()
    pltpu.sync_copy(acc.at[pl.ds(r_base, _RPT), :],
                    out_hbm.at[c, pl.ds(r_base, _RPT), :])

    @pl.when(s == _NS - 1)
    def _():
        pltpu.sync_copy(acc.at[pl.ds(_NS * _RPT, _N - _NS * _RPT), :],
                        out_hbm.at[c, pl.ds(_NS * _RPT, _N - _NS * _RPT), :])


_BE = 2000


def _gate_body(a_ref, ea_ref, wef_ref, wes_ref, bf_ref, bs_ref, m_ref):
    a = a_ref[...]
    ea = ea_ref[...]
    zf = a[:, :_D] + jnp.dot(ea, wef_ref[...],
                             preferred_element_type=jnp.float32) + bf_ref[...]
    zs = a[:, _D:] + jnp.dot(ea, wes_ref[...],
                             preferred_element_type=jnp.float32) + bs_ref[...]
    m_ref[...] = jax.nn.sigmoid(zf) * jax.nn.softplus(zs)


def _gate(a, ea, wef, wes, bf, bs):
    grid = (_E // _BE,)
    return pl.pallas_call(
        _gate_body,
        grid=grid,
        in_specs=[
            pl.BlockSpec((_BE, 2 * _D), lambda i: (i, 0)),
            pl.BlockSpec((_BE, _ED), lambda i: (i, 0)),
            pl.BlockSpec((_ED, _D), lambda i: (0, 0)),
            pl.BlockSpec((_ED, _D), lambda i: (0, 0)),
            pl.BlockSpec((1, _D), lambda i: (0, 0)),
            pl.BlockSpec((1, _D), lambda i: (0, 0)),
        ],
        out_specs=pl.BlockSpec((_BE, _D), lambda i: (i, 0)),
        out_shape=jax.ShapeDtypeStruct((_E, _D), jnp.float32),
    )(a, ea, wef, wes, bf, bs)


_BN_ROWS = 2000


def _proj_body(h_ref, wd_ref, ws_ref, pd_ref, ps_ref):
    h = h_ref[...]
    pd_ref[...] = jnp.dot(h, wd_ref[...], preferred_element_type=jnp.float32)
    ps_ref[...] = jnp.dot(h, ws_ref[...], preferred_element_type=jnp.float32)


def _projs(h, wd, ws):
    grid = (_N // _BN_ROWS,)
    return pl.pallas_call(
        _proj_body,
        grid=grid,
        in_specs=[
            pl.BlockSpec((_BN_ROWS, _D), lambda i: (i, 0)),
            pl.BlockSpec((_D, 2 * _D), lambda i: (0, 0)),
            pl.BlockSpec((_D, 2 * _D), lambda i: (0, 0)),
        ],
        out_specs=[
            pl.BlockSpec((_BN_ROWS, 2 * _D), lambda i: (i, 0)),
            pl.BlockSpec((_BN_ROWS, 2 * _D), lambda i: (i, 0)),
        ],
        out_shape=[
            jax.ShapeDtypeStruct((_N, 2 * _D), jnp.float32),
            jax.ShapeDtypeStruct((_N, 2 * _D), jnp.float32),
        ],
    )(h, wd, ws)


def _bn(h, g, b):
    mu = jnp.mean(h, axis=0, keepdims=True)
    var = jnp.mean((h - mu) * (h - mu), axis=0, keepdims=True)
    return (h - mu) / jnp.sqrt(var + 1e-5) * g + b


def _bnproj_body(p_ref, h_ref, g_ref, b_ref, wd_ref, ws_ref,
                 hn_ref, pd_ref, ps_ref):
    agg = p_ref[0] + p_ref[1]
    hn = _bn(agg, g_ref[...], b_ref[...]) + h_ref[...]
    hn_ref[...] = hn
    pd_ref[...] = jnp.dot(hn, wd_ref[...], preferred_element_type=jnp.float32)
    ps_ref[...] = jnp.dot(hn, ws_ref[...], preferred_element_type=jnp.float32)


def _bnproj(parts, h, g, b, wd, ws):
    return pl.pallas_call(
        _bnproj_body,
        in_specs=[
            pl.BlockSpec((2, _N, _D), lambda: (0, 0, 0)),
            pl.BlockSpec((_N, _D), lambda: (0, 0)),
            pl.BlockSpec((1, _D), lambda: (0, 0)),
            pl.BlockSpec((1, _D), lambda: (0, 0)),
            pl.BlockSpec((_D, 2 * _D), lambda: (0, 0)),
            pl.BlockSpec((_D, 2 * _D), lambda: (0, 0)),
        ],
        out_specs=[
            pl.BlockSpec((_N, _D), lambda: (0, 0)),
            pl.BlockSpec((_N, 2 * _D), lambda: (0, 0)),
            pl.BlockSpec((_N, 2 * _D), lambda: (0, 0)),
        ],
        out_shape=[
            jax.ShapeDtypeStruct((_N, _D), jnp.float32),
            jax.ShapeDtypeStruct((_N, 2 * _D), jnp.float32),
            jax.ShapeDtypeStruct((_N, 2 * _D), jnp.float32),
        ],
    )(parts, h, g, b, wd, ws)


def _head_body(p_ref, h_ref, g_ref, b_ref, batch_ref,
               w1_ref, b1_ref, g1_ref, be1_ref,
               w2_ref, b2_ref, g2_ref, be2_ref,
               w3_ref, b3_ref, out_ref):
    agg = p_ref[0] + p_ref[1]
    h4 = _bn(agg, g_ref[...], b_ref[...]) + h_ref[...]
    bvec = batch_ref[...]
    gids = lax.broadcasted_iota(jnp.int32, (_G, 1), 0)
    onehot = (bvec == gids).astype(jnp.float32)
    sums = jnp.dot(onehot, h4, preferred_element_type=jnp.float32, precision=lax.Precision.HIGHEST)
    cnt = jnp.sum(onehot, axis=1, keepdims=True)
    pooled = sums / jnp.maximum(cnt, 1.0)
    z1 = jnp.dot(pooled, w1_ref[...], preferred_element_type=jnp.float32, precision=lax.Precision.HIGHEST) + b1_ref[...]
    a1 = jnp.maximum(_bn(z1, g1_ref[...], be1_ref[...]), 0.0)
    z2 = jnp.dot(a1, w2_ref[...], preferred_element_type=jnp.float32, precision=lax.Precision.HIGHEST) + b2_ref[...]
    a2 = jnp.maximum(_bn(z2, g2_ref[...], be2_ref[...]), 0.0)
    out_ref[...] = jnp.dot(a2, w3_ref[...], preferred_element_type=jnp.float32, precision=lax.Precision.HIGHEST) + b3_ref[...]


def _head(parts, h, g, b, batch2d, w1, b1, g1, be1, w2, b2, g2, be2, w3, b3):
    full = lambda s: pl.BlockSpec(s, lambda: tuple(0 for _ in s))
    return pl.pallas_call(
        _head_body,
        in_specs=[
            pl.BlockSpec((2, _N, _D), lambda: (0, 0, 0)),
            full((_N, _D)), full((1, _D)), full((1, _D)), full((1, _N)),
            full((_D, _H1)), full((1, _H1)), full((1, _H1)), full((1, _H1)),
            full((_H1, _H2)), full((1, _H2)), full((1, _H2)), full((1, _H2)),
            full((_H2, _OUT)), full((1, _OUT)),
        ],
        out_specs=full((_G, _OUT)),
        out_shape=jax.ShapeDtypeStruct((_G, _OUT), jnp.float32),
    )(parts, h, g, b,
      batch2d, w1, b1, g1, be1, w2, b2, g2, be2, w3, b3)


def kernel(x, edge_index, edge_attr, batch, Wf, bf, Ws, bs, bng, bnb,
           W1, b1, g1, be1, W2, b2, g2, be2, W3, b3):
    src3 = edge_index[0].reshape(_NW, _GNC, _GCH)
    dst3 = edge_index[1].reshape(_NW, _GNC, _GCH)
    dsts = edge_index[1].reshape(_NW, _EPW // _SIR, _SIR)
    zeros = jnp.zeros((_N, _D), jnp.float32)
    batch2d = batch.reshape(1, _N)

    wd = [jnp.concatenate([Wf[i, :_D], Ws[i, :_D]], axis=1) for i in range(4)]
    wsrc = [jnp.concatenate([Wf[i, _D:2 * _D], Ws[i, _D:2 * _D]], axis=1)
            for i in range(4)]
    wef = [Wf[i, 2 * _D:] for i in range(4)]
    wes = [Ws[i, 2 * _D:] for i in range(4)]
    bf2 = [bf[i].reshape(1, _D) for i in range(4)]
    bs2 = [bs[i].reshape(1, _D) for i in range(4)]
    g2d = [bng[i].reshape(1, _D) for i in range(4)]
    b2d = [bnb[i].reshape(1, _D) for i in range(4)]

    h = x
    pd, ps = _projs(x, wd[0], wsrc[0])
    out = None
    for i in range(4):
        a = _sc_gather(pd, ps, src3, dst3)
        m = _gate(a, edge_attr, wef[i], wes[i], bf2[i], bs2[i])
        parts = _sc_scatter(m, dsts, zeros)
        if i < 3:
            h, pd, ps = _bnproj(parts, h, g2d[i], b2d[i], wd[i + 1], wsrc[i + 1])
        else:
            out = _head(parts, h, g2d[i], b2d[i], batch2d,
                        W1, b1.reshape(1, _H1), g1.reshape(1, _H1),
                        be1.reshape(1, _H1),
                        W2, b2.reshape(1, _H2), g2.reshape(1, _H2),
                        be2.reshape(1, _H2),
                        W3, b3.reshape(1, _OUT))
    return out


# speedup vs baseline: 1.2734x; 1.1317x over previous
import functools

import jax
import jax.numpy as jnp
from jax import lax
from jax.experimental import pallas as pl
from jax.experimental.pallas import tpu as pltpu
from jax.experimental.pallas import tpu_sc as plsc

_N, _E, _D, _ED, _G = 10000, 320000, 128, 16, 64
_H1, _H2, _OUT = 256, 128, 1
_NC, _NS, _L = 2, 16, 16
_NW = _NC * _NS
_EPW = _E // _NW
_CH = 80
_NCHK = _EPW // _CH
_RPT = 624
_ZR = 208
_GCH = 80
_GNC = _EPW // _GCH
_SCH = 40
_SNC = _EPW // _SCH
_SIR = _SCH

_mesh = plsc.VectorSubcoreMesh(core_axis_name="c", subcore_axis_name="s")


_GCH = 40
_GNC = _EPW // _GCH


@functools.partial(
    pl.kernel,
    out_type=(jax.ShapeDtypeStruct((_E, _D), jnp.int32),
              jax.ShapeDtypeStruct((_E, _D), jnp.int32)),
    mesh=_mesh,
    scratch_types=[
        pltpu.VMEM((_GNC, _GCH), jnp.int32),
        pltpu.VMEM((_GNC, _GCH), jnp.int32),
        pltpu.VMEM((_GCH, _D), jnp.int32),
        pltpu.VMEM((_GCH, _D), jnp.int32),
        pltpu.VMEM((_GCH, _D), jnp.int32),
        pltpu.VMEM((_GCH, _D), jnp.int32),
        pltpu.VMEM((_GCH, _D), jnp.int32),
        pltpu.VMEM((_GCH, _D), jnp.int32),
        pltpu.SemaphoreType.DMA,
        pltpu.SemaphoreType.DMA,
        pltpu.SemaphoreType.DMA,
        pltpu.SemaphoreType.DMA,
        pltpu.SemaphoreType.DMA,
        pltpu.SemaphoreType.DMA,
    ],
)
def _sc_gather(pd_hbm, ps_hbm, src3_hbm, dst3_hbm, ad_hbm, as_hbm,
               di2, si2, gd0, gs0, gd1, gs1, gd2, gs2,
               sg0, sg1, sg2, sw0, sw1, sw2):
    c = lax.axis_index("c")
    s = lax.axis_index("s")
    wid = s * _NC + c
    base = wid * _EPW
    gd = (gd0, gd1, gd2)
    gs = (gs0, gs1, gs2)
    sg = (sg0, sg1, sg2)
    sw = (sw0, sw1, sw2)

    pltpu.sync_copy(dst3_hbm.at[wid], di2)
    pltpu.sync_copy(src3_hbm.at[wid], si2)

    def start_g(i, b):
        pltpu.async_copy(pd_hbm.at[di2.at[i]], gd[b], sg[b])
        pltpu.async_copy(ps_hbm.at[si2.at[i]], gs[b], sg[b])

    def wait_g(b):
        pltpu.make_async_copy(pd_hbm.at[di2.at[0]], gd[b], sg[b]).wait()
        pltpu.make_async_copy(ps_hbm.at[si2.at[0]], gs[b], sg[b]).wait()

    def start_wb(i, b):
        sl = pl.ds(base + i * _GCH, _GCH)
        pltpu.async_copy(gd[b], ad_hbm.at[sl, :], sw[b])
        pltpu.async_copy(gs[b], as_hbm.at[sl, :], sw[b])

    def wait_wb(b):
        sl = pl.ds(base, _GCH)
        pltpu.make_async_copy(gd[b], ad_hbm.at[sl, :], sw[b]).wait()
        pltpu.make_async_copy(gs[b], as_hbm.at[sl, :], sw[b]).wait()

    start_g(0, 0)
    start_g(1, 1)

    @pl.loop(0, (_GNC - 1) // 3)
    def _triple(p):
        for b in (0, 1, 2):
            i = 3 * p + b
            wait_g(b)
            start_wb(i, b)
            b2 = (b + 2) % 3
            if b == 0:
                @pl.when(p > 0)
                def _():
                    wait_wb(b2)
                    start_g(i + 2, b2)

                @pl.when(p == 0)
                def _():
                    start_g(i + 2, b2)
            else:
                wait_wb(b2)

                @pl.when(i + 2 < _GNC)
                def _():
                    start_g(i + 2, b2)

    wait_g(0)
    start_wb(_GNC - 1, 0)
    wait_wb(2)
    wait_wb(0)


@functools.partial(
    pl.kernel,
    out_type=jax.ShapeDtypeStruct((_NC, _N, _D), jnp.float32),
    mesh=_mesh,
    scratch_types=[
        pltpu.VMEM((_EPW // _SIR, _SIR), jnp.int32),
        pltpu.VMEM((_SCH, _D), jnp.float32),
        pltpu.VMEM((_SCH, _D), jnp.float32),
        pltpu.VMEM_SHARED((_N, _D), jnp.float32),
        pltpu.SemaphoreType.DMA,
        pltpu.SemaphoreType.DMA,
        pltpu.SemaphoreType.DMA,
        pltpu.SemaphoreType.DMA,
    ],
)
def _sc_scatter(m_hbm, dsts_hbm, zeros_hbm, out_hbm, di2, mb0, mb1, acc,
                sm0, sm1, ssc0, ssc1):
    c = lax.axis_index("c")
    s = lax.axis_index("s")
    wid = s * _NC + c
    base = wid * _EPW
    mb = (mb0, mb1)
    sm = (sm0, sm1)
    ssc = (ssc0, ssc1)

    pltpu.sync_copy(dsts_hbm.at[wid], di2)

    r_base = s * _RPT
    pltpu.sync_copy(zeros_hbm.at[pl.ds(r_base, _RPT), :],
                    acc.at[pl.ds(r_base, _RPT), :])

    @pl.when(s == _NS - 1)
    def _():
        pltpu.sync_copy(zeros_hbm.at[pl.ds(_NS * _RPT, _N - _NS * _RPT), :],
                        acc.at[pl.ds(_NS * _RPT, _N - _NS * _RPT), :])

    plsc.subcore_barrier()

    def start_m(i, b):
        pltpu.async_copy(m_hbm.at[pl.ds(base + i * _SCH, _SCH), :], mb[b], sm[b])

    def wait_m(b):
        pltpu.make_async_copy(m_hbm.at[pl.ds(base, _SCH), :], mb[b], sm[b]).wait()

    def start_sc(i, b):
        pltpu.async_copy(mb[b], acc.at[di2.at[i]], ssc[b], add=True)

    def wait_sc(b):
        pltpu.make_async_copy(mb[b], acc.at[di2.at[0]], ssc[b]).wait()

    start_m(0, 0)
    start_m(1, 1)

    @pl.loop(0, _SNC // 2)
    def _pair(p):
        for b in (0, 1):
            wait_m(b)
            start_sc(2 * p + b, b)
        for b in (0, 1):
            i = 2 * p + b
            wait_sc(b)

            @pl.when(i + 2 < _SNC)
            def _():
                start_m(i + 2, b)

    plsc.subcore_barrier()
    pltpu.sync_copy(acc.at[pl.ds(r_base, _RPT), :],
                    out_hbm.at[c, pl.ds(r_base, _RPT), :])

    @pl.when(s == _NS - 1)
    def _():
        pltpu.sync_copy(acc.at[pl.ds(_NS * _RPT, _N - _NS * _RPT), :],
                        out_hbm.at[c, pl.ds(_NS * _RPT, _N - _NS * _RPT), :])


_BE = 2000


def _unpack_lo(u):
    return lax.bitcast_convert_type(lax.shift_left(u, 16), jnp.float32)


def _unpack_hi(u):
    return lax.bitcast_convert_type(
        lax.bitwise_and(u, jnp.int32(-65536)), jnp.float32)


def _gate_body(ad_ref, as_ref, ea_ref, wef_ref, wes_ref, bf_ref, bs_ref,
               m_ref):
    ud = ad_ref[...]
    us = as_ref[...]
    ea = ea_ref[...]
    zf = _unpack_lo(ud) + _unpack_lo(us) + jnp.dot(
        ea, wef_ref[...], preferred_element_type=jnp.float32) + bf_ref[...]
    zs = _unpack_hi(ud) + _unpack_hi(us) + jnp.dot(
        ea, wes_ref[...], preferred_element_type=jnp.float32) + bs_ref[...]
    m_ref[...] = jax.nn.sigmoid(zf) * jax.nn.softplus(zs)


def _gate(ad, as_, ea, wef, wes, bf, bs):
    grid = (_E // _BE,)
    return pl.pallas_call(
        _gate_body,
        grid=grid,
        in_specs=[
            pl.BlockSpec((_BE, _D), lambda i: (i, 0)),
            pl.BlockSpec((_BE, _D), lambda i: (i, 0)),
            pl.BlockSpec((_BE, _ED), lambda i: (i, 0)),
            pl.BlockSpec((_ED, _D), lambda i: (0, 0)),
            pl.BlockSpec((_ED, _D), lambda i: (0, 0)),
            pl.BlockSpec((1, _D), lambda i: (0, 0)),
            pl.BlockSpec((1, _D), lambda i: (0, 0)),
        ],
        out_specs=pl.BlockSpec((_BE, _D), lambda i: (i, 0)),
        out_shape=jax.ShapeDtypeStruct((_E, _D), jnp.float32),
    )(ad, as_, ea, wef, wes, bf, bs)


_BN_ROWS = 2000


def _pack_bf16_pair(zf, zs):
    uf = lax.bitcast_convert_type(zf, jnp.int32)
    us = lax.bitcast_convert_type(zs, jnp.int32)
    one = jnp.int32(1)
    rf = lax.shift_right_logical(
        uf + jnp.int32(0x7FFF)
        + lax.bitwise_and(lax.shift_right_logical(uf, 16), one), 16)
    rs = lax.shift_right_logical(
        us + jnp.int32(0x7FFF)
        + lax.bitwise_and(lax.shift_right_logical(us, 16), one), 16)
    return lax.bitwise_or(lax.shift_left(rs, 16), rf)


def _proj_body(h_ref, wdf_ref, wds_ref, wsf_ref, wss_ref, pd_ref, ps_ref):
    h = h_ref[...]
    pd_ref[...] = _pack_bf16_pair(
        jnp.dot(h, wdf_ref[...], preferred_element_type=jnp.float32),
        jnp.dot(h, wds_ref[...], preferred_element_type=jnp.float32))
    ps_ref[...] = _pack_bf16_pair(
        jnp.dot(h, wsf_ref[...], preferred_element_type=jnp.float32),
        jnp.dot(h, wss_ref[...], preferred_element_type=jnp.float32))


def _projs(h, wdf, wds, wsf, wss):
    grid = (_N // _BN_ROWS,)
    return pl.pallas_call(
        _proj_body,
        grid=grid,
        in_specs=[
            pl.BlockSpec((_BN_ROWS, _D), lambda i: (i, 0)),
            pl.BlockSpec((_D, _D), lambda i: (0, 0)),
            pl.BlockSpec((_D, _D), lambda i: (0, 0)),
            pl.BlockSpec((_D, _D), lambda i: (0, 0)),
            pl.BlockSpec((_D, _D), lambda i: (0, 0)),
        ],
        out_specs=[
            pl.BlockSpec((_BN_ROWS, _D), lambda i: (i, 0)),
            pl.BlockSpec((_BN_ROWS, _D), lambda i: (i, 0)),
        ],
        out_shape=[
            jax.ShapeDtypeStruct((_N, _D), jnp.int32),
            jax.ShapeDtypeStruct((_N, _D), jnp.int32),
        ],
    )(h, wdf, wds, wsf, wss)


def _bn(h, g, b):
    mu = jnp.mean(h, axis=0, keepdims=True)
    var = jnp.mean((h - mu) * (h - mu), axis=0, keepdims=True)
    return (h - mu) / jnp.sqrt(var + 1e-5) * g + b


def _bnproj_body(p_ref, h_ref, g_ref, b_ref, wdf_ref, wds_ref, wsf_ref,
                 wss_ref, hn_ref, pd_ref, ps_ref):
    agg = p_ref[0] + p_ref[1]
    hn = _bn(agg, g_ref[...], b_ref[...]) + h_ref[...]
    hn_ref[...] = hn
    pd_ref[...] = _pack_bf16_pair(
        jnp.dot(hn, wdf_ref[...], preferred_element_type=jnp.float32),
        jnp.dot(hn, wds_ref[...], preferred_element_type=jnp.float32))
    ps_ref[...] = _pack_bf16_pair(
        jnp.dot(hn, wsf_ref[...], preferred_element_type=jnp.float32),
        jnp.dot(hn, wss_ref[...], preferred_element_type=jnp.float32))


def _bnproj(parts, h, g, b, wdf, wds, wsf, wss):
    return pl.pallas_call(
        _bnproj_body,
        in_specs=[
            pl.BlockSpec((2, _N, _D), lambda: (0, 0, 0)),
            pl.BlockSpec((_N, _D), lambda: (0, 0)),
            pl.BlockSpec((1, _D), lambda: (0, 0)),
            pl.BlockSpec((1, _D), lambda: (0, 0)),
            pl.BlockSpec((_D, _D), lambda: (0, 0)),
            pl.BlockSpec((_D, _D), lambda: (0, 0)),
            pl.BlockSpec((_D, _D), lambda: (0, 0)),
            pl.BlockSpec((_D, _D), lambda: (0, 0)),
        ],
        out_specs=[
            pl.BlockSpec((_N, _D), lambda: (0, 0)),
            pl.BlockSpec((_N, _D), lambda: (0, 0)),
            pl.BlockSpec((_N, _D), lambda: (0, 0)),
        ],
        out_shape=[
            jax.ShapeDtypeStruct((_N, _D), jnp.float32),
            jax.ShapeDtypeStruct((_N, _D), jnp.int32),
            jax.ShapeDtypeStruct((_N, _D), jnp.int32),
        ],
    )(parts, h, g, b, wdf, wds, wsf, wss)


def _head_body(p_ref, h_ref, g_ref, b_ref, batch_ref,
               w1_ref, b1_ref, g1_ref, be1_ref,
               w2_ref, b2_ref, g2_ref, be2_ref,
               w3_ref, b3_ref, out_ref):
    agg = p_ref[0] + p_ref[1]
    h4 = _bn(agg, g_ref[...], b_ref[...]) + h_ref[...]
    bvec = batch_ref[...]
    gids = lax.broadcasted_iota(jnp.int32, (_G, 1), 0)
    onehot = (bvec == gids).astype(jnp.float32)
    sums = jnp.dot(onehot, h4, preferred_element_type=jnp.float32, precision=lax.Precision.HIGHEST)
    cnt = jnp.sum(onehot, axis=1, keepdims=True)
    pooled = sums / jnp.maximum(cnt, 1.0)
    z1 = jnp.dot(pooled, w1_ref[...], preferred_element_type=jnp.float32, precision=lax.Precision.HIGHEST) + b1_ref[...]
    a1 = jnp.maximum(_bn(z1, g1_ref[...], be1_ref[...]), 0.0)
    z2 = jnp.dot(a1, w2_ref[...], preferred_element_type=jnp.float32, precision=lax.Precision.HIGHEST) + b2_ref[...]
    a2 = jnp.maximum(_bn(z2, g2_ref[...], be2_ref[...]), 0.0)
    out_ref[...] = jnp.dot(a2, w3_ref[...], preferred_element_type=jnp.float32, precision=lax.Precision.HIGHEST) + b3_ref[...]


def _head(parts, h, g, b, batch2d, w1, b1, g1, be1, w2, b2, g2, be2, w3, b3):
    full = lambda s: pl.BlockSpec(s, lambda: tuple(0 for _ in s))
    return pl.pallas_call(
        _head_body,
        in_specs=[
            pl.BlockSpec((2, _N, _D), lambda: (0, 0, 0)),
            full((_N, _D)), full((1, _D)), full((1, _D)), full((1, _N)),
            full((_D, _H1)), full((1, _H1)), full((1, _H1)), full((1, _H1)),
            full((_H1, _H2)), full((1, _H2)), full((1, _H2)), full((1, _H2)),
            full((_H2, _OUT)), full((1, _OUT)),
        ],
        out_specs=full((_G, _OUT)),
        out_shape=jax.ShapeDtypeStruct((_G, _OUT), jnp.float32),
    )(parts, h, g, b,
      batch2d, w1, b1, g1, be1, w2, b2, g2, be2, w3, b3)


def kernel(x, edge_index, edge_attr, batch, Wf, bf, Ws, bs, bng, bnb,
           W1, b1, g1, be1, W2, b2, g2, be2, W3, b3):
    src3 = edge_index[0].reshape(_NW, _GNC, _GCH)
    dst3 = edge_index[1].reshape(_NW, _GNC, _GCH)
    dsts = edge_index[1].reshape(_NW, _EPW // _SIR, _SIR)
    zeros = jnp.zeros((_N, _D), jnp.float32)
    batch2d = batch.reshape(1, _N)

    wdf = [Wf[i, :_D] for i in range(4)]
    wds = [Ws[i, :_D] for i in range(4)]
    wsf = [Wf[i, _D:2 * _D] for i in range(4)]
    wss = [Ws[i, _D:2 * _D] for i in range(4)]
    wef = [Wf[i, 2 * _D:] for i in range(4)]
    wes = [Ws[i, 2 * _D:] for i in range(4)]
    bf2 = [bf[i].reshape(1, _D) for i in range(4)]
    bs2 = [bs[i].reshape(1, _D) for i in range(4)]
    g2d = [bng[i].reshape(1, _D) for i in range(4)]
    b2d = [bnb[i].reshape(1, _D) for i in range(4)]

    h = x
    pd, ps = _projs(x, wdf[0], wds[0], wsf[0], wss[0])
    out = None
    for i in range(4):
        ad, as_ = _sc_gather(pd, ps, src3, dst3)
        m = _gate(ad, as_, edge_attr, wef[i], wes[i], bf2[i], bs2[i])
        parts = _sc_scatter(m, dsts, zeros)
        if i < 3:
            h, pd, ps = _bnproj(parts, h, g2d[i], b2d[i],
                                wdf[i + 1], wds[i + 1], wsf[i + 1], wss[i + 1])
        else:
            out = _head(parts, h, g2d[i], b2d[i], batch2d,
                        W1, b1.reshape(1, _H1), g1.reshape(1, _H1),
                        be1.reshape(1, _H1),
                        W2, b2.reshape(1, _H2), g2.reshape(1, _H2),
                        be2.reshape(1, _H2),
                        W3, b3.reshape(1, _OUT))
    return out
